# Initial kernel scaffold; baseline (speedup 1.0000x reference)
import jax
import jax.numpy as jnp
from jax.experimental import pallas as pl


def kernel(node_feats, node_lens, token_table):
    raise NotImplementedError("write your pallas kernel here")


# speedup vs baseline: 1.5550x; 1.5550x over previous
import functools

import jax
import jax.numpy as jnp
from jax import lax
from jax.experimental import pallas as pl
from jax.experimental.pallas import tpu as pltpu
from jax.experimental.pallas import tpu_sc as plsc

VOCAB = 1000000
EMB = 32
BATCH = 4096
HIST = 200

NC = 2
NS = 16
NW = NC * NS

B = BATCH * HIST
IDXW = 128
ROWS_PER_W = B // NW
IROWS_PER_W = ROWS_PER_W // IDXW
BLK_IROWS = 8
BLK = BLK_IROWS * IDXW
NBLK = IROWS_PER_W // BLK_IROWS


def _body(table_hbm, idx_hbm, out_hbm, idx_v, rows_v, sem):
    wid = lax.axis_index("s") * NC + lax.axis_index("c")
    irow_base = wid * IROWS_PER_W
    row_base = wid * ROWS_PER_W
    pltpu.sync_copy(idx_hbm.at[pl.ds(irow_base, IROWS_PER_W)], idx_v)

    def block(g, carry):
        cps = []
        for j in range(BLK_IROWS):
            cp = pltpu.async_copy(
                table_hbm.at[idx_v.at[g * BLK_IROWS + j]],
                rows_v.at[pl.ds(j * IDXW, IDXW)],
                sem,
            )
            cps.append(cp)
        for cp in cps:
            cp.wait()
        pltpu.sync_copy(rows_v, out_hbm.at[pl.ds(row_base + g * BLK, BLK)])
        return carry

    lax.fori_loop(0, NBLK, block, 0)


@jax.jit
def _gather(token_table, idx):
    mesh = plsc.VectorSubcoreMesh(core_axis_name="c", subcore_axis_name="s")
    f = pl.kernel(
        _body,
        out_type=jax.ShapeDtypeStruct((B, EMB), jnp.float32),
        mesh=mesh,
        scratch_types=[
            pltpu.VMEM((IROWS_PER_W, IDXW), jnp.int32),
            pltpu.VMEM((BLK, EMB), jnp.float32),
            pltpu.SemaphoreType.DMA,
        ],
        compiler_params=pltpu.CompilerParams(use_tc_tiling_on_sc=False),
    )
    return f(token_table, idx)


def kernel(node_feats, node_lens, token_table):
    del node_lens
    idx = node_feats.reshape(B // IDXW, IDXW).astype(jnp.int32)
    out = _gather(token_table, idx)
    return out.reshape(BATCH, HIST, EMB)


# speedup vs baseline: 1.5710x; 1.0103x over previous
import jax
import jax.numpy as jnp
from jax import lax
from jax.experimental import pallas as pl
from jax.experimental.pallas import tpu as pltpu
from jax.experimental.pallas import tpu_sc as plsc

VOCAB = 1000000
EMB = 32
BATCH = 4096
HIST = 200

NC = 2
NS = 16
NW = NC * NS

B = BATCH * HIST
IDXW = 128
ROWS_PER_W = B // NW
IROWS_PER_W = ROWS_PER_W // IDXW
BLK_IROWS = 10
BLK = BLK_IROWS * IDXW
NBLK = IROWS_PER_W // BLK_IROWS


def _body(table_hbm, idx_hbm, out_hbm, idx_v, buf0, buf1,
          gsem0, gsem1, osem0, osem1):
    wid = lax.axis_index("s") * NC + lax.axis_index("c")
    irow_base = wid * IROWS_PER_W
    row_base = wid * ROWS_PER_W
    pltpu.sync_copy(idx_hbm.at[pl.ds(irow_base, IROWS_PER_W)], idx_v)

    bufs = (buf0, buf1)
    gsems = (gsem0, gsem1)
    osems = (osem0, osem1)

    def fire_gathers(g, b):
        for j in range(BLK_IROWS):
            pltpu.async_copy(
                table_hbm.at[idx_v.at[g * BLK_IROWS + j]],
                bufs[b].at[pl.ds(j * IDXW, IDXW)],
                gsems[b],
            )

    def wait_gathers(b):
        pltpu.make_async_copy(out_hbm.at[pl.ds(0, BLK)], bufs[b],
                              gsems[b]).wait()

    def wait_outcopy(b):
        pltpu.make_async_copy(bufs[b], out_hbm.at[pl.ds(0, BLK)],
                              osems[b]).wait()

    fire_gathers(0, 0)
    fire_gathers(1, 1)

    def step(i, carry):
        g = 2 * i
        wait_gathers(0)
        pltpu.async_copy(buf0, out_hbm.at[pl.ds(row_base + g * BLK, BLK)],
                         osem0)
        wait_gathers(1)
        pltpu.async_copy(buf1, out_hbm.at[pl.ds(row_base + (g + 1) * BLK, BLK)],
                         osem1)
        wait_outcopy(0)

        @pl.when(g + 2 < NBLK)
        def _():
            fire_gathers(g + 2, 0)

        wait_outcopy(1)

        @pl.when(g + 3 < NBLK)
        def _():
            fire_gathers(g + 3, 1)

        return carry

    lax.fori_loop(0, NBLK // 2, step, 0)


@jax.jit
def _gather(token_table, idx):
    mesh = plsc.VectorSubcoreMesh(core_axis_name="c", subcore_axis_name="s")
    f = pl.kernel(
        _body,
        out_type=jax.ShapeDtypeStruct((B, EMB), jnp.float32),
        mesh=mesh,
        scratch_types=[
            pltpu.VMEM((IROWS_PER_W, IDXW), jnp.int32),
            pltpu.VMEM((BLK, EMB), jnp.float32),
            pltpu.VMEM((BLK, EMB), jnp.float32),
            pltpu.SemaphoreType.DMA,
            pltpu.SemaphoreType.DMA,
            pltpu.SemaphoreType.DMA,
            pltpu.SemaphoreType.DMA,
        ],
        compiler_params=pltpu.CompilerParams(use_tc_tiling_on_sc=False),
    )
    return f(token_table, idx)


def kernel(node_feats, node_lens, token_table):
    del node_lens
    idx = node_feats.reshape(B // IDXW, IDXW).astype(jnp.int32)
    out = _gather(token_table, idx)
    return out.reshape(BATCH, HIST, EMB)
